# baseline (speedup 1.0000x reference)
import jax
import jax.numpy as jnp
from jax.experimental import pallas as pl
from jax.experimental.pallas import tpu as pltpu

_ROWS = 100000
_COLS = 256
_CHUNKS = 10
_CHUNK_ROWS = _ROWS // _CHUNKS


def _copy_body(src_ref, dst_ref, sem):
    for i in range(_CHUNKS):
        pltpu.make_async_copy(
            src_ref.at[pl.ds(i * _CHUNK_ROWS, _CHUNK_ROWS), :],
            dst_ref.at[pl.ds(i * _CHUNK_ROWS, _CHUNK_ROWS), :],
            sem.at[i],
        ).start()
    for i in range(_CHUNKS):
        pltpu.make_async_copy(
            src_ref.at[pl.ds(i * _CHUNK_ROWS, _CHUNK_ROWS), :],
            dst_ref.at[pl.ds(i * _CHUNK_ROWS, _CHUNK_ROWS), :],
            sem.at[i],
        ).wait()


def kernel(weight):
    return pl.pallas_call(
        _copy_body,
        in_specs=[pl.BlockSpec(memory_space=pltpu.MemorySpace.HBM)],
        out_specs=pl.BlockSpec(memory_space=pltpu.MemorySpace.HBM),
        out_shape=jax.ShapeDtypeStruct((_ROWS, _COLS), jnp.float32),
        scratch_shapes=[pltpu.SemaphoreType.DMA((_CHUNKS,))],
    )(weight)


# speedup vs baseline: 47.7419x; 47.7419x over previous
import jax
import jax.numpy as jnp
from jax.experimental import pallas as pl
from jax.experimental.pallas import tpu as pltpu

_ROWS = 100000
_COLS = 256
_BLOCK_ROWS = 4000


def _copy_body(src_ref, dst_ref):
    dst_ref[...] = src_ref[...]


def kernel(weight):
    n_blocks = _ROWS // _BLOCK_ROWS
    return pl.pallas_call(
        _copy_body,
        grid=(n_blocks,),
        in_specs=[pl.BlockSpec((_BLOCK_ROWS, _COLS), lambda i: (i, 0))],
        out_specs=pl.BlockSpec((_BLOCK_ROWS, _COLS), lambda i: (i, 0)),
        out_shape=jax.ShapeDtypeStruct((_ROWS, _COLS), jnp.float32),
    )(weight)


# speedup vs baseline: 49.1570x; 1.0296x over previous
import jax
import jax.numpy as jnp
from jax.experimental import pallas as pl
from jax.experimental.pallas import tpu as pltpu

_ROWS = 100000
_COLS = 256
_BLOCK_ROWS = 10000


def _copy_body(src_ref, dst_ref):
    dst_ref[...] = src_ref[...]


def kernel(weight):
    n_blocks = _ROWS // _BLOCK_ROWS
    return pl.pallas_call(
        _copy_body,
        grid=(n_blocks,),
        in_specs=[pl.BlockSpec((_BLOCK_ROWS, _COLS), lambda i: (i, 0))],
        out_specs=pl.BlockSpec((_BLOCK_ROWS, _COLS), lambda i: (i, 0)),
        out_shape=jax.ShapeDtypeStruct((_ROWS, _COLS), jnp.float32),
    )(weight)
